# baseline (speedup 1.0000x reference)
import jax
import jax.numpy as jnp
from jax import lax
from jax.experimental import pallas as pl
from jax.experimental.pallas import tpu as pltpu

_RB = 512


def _gcn_body(a_ref, x_ref, w_ref, b_ref, o_ref, tri_ref, d_ref, ybf_ref, acc_ref):
    k = pl.program_id(0)
    ns = d_ref.shape[0]
    off = [_RB * c * (c + 1) // 2 for c in range(ns + 1)]

    for c in range(ns):
        @pl.when(k == c + 1)
        def _stage_a(c=c):
            a = a_ref[...]
            deg = jnp.sum(a, axis=1, keepdims=True) + 1.0
            d = lax.rsqrt(deg)
            d_ref[pl.ds(c, 1)] = d[None]
            xw = lax.dot_general(
                x_ref[...], w_ref[...],
                dimension_numbers=(((1,), (1,)), ((), ())),
                preferred_element_type=jnp.float32,
            )
            ybf_ref[pl.ds(c * _RB, _RB), :] = (d * xw).astype(jnp.bfloat16)

            for j in range(c, ns):
                tri_ref[off[j] + c * _RB:off[j] + (c + 1) * _RB, :] = (
                    a[:, j * _RB:(j + 1) * _RB].astype(jnp.bfloat16))

            if c > 0:
                lo = a[:, 0:c * _RB].astype(jnp.bfloat16)
                z1 = lax.dot_general(
                    lo, ybf_ref[0:c * _RB, :],
                    dimension_numbers=(((1,), (0,)), ((), ())),
                    preferred_element_type=jnp.float32,
                )
                acc_ref[pl.ds(c * _RB, _RB), :] = z1
            else:
                acc_ref[pl.ds(0, _RB), :] = jnp.zeros(
                    (_RB, acc_ref.shape[1]), jnp.float32)

    for c in range(ns):
        @pl.when(k == c + 2)
        def _stage_b(c=c):
            rows = (c + 1) * _RB
            yc = ybf_ref[pl.ds(c * _RB, _RB), :]
            z2 = lax.dot_general(
                tri_ref[off[c]:off[c] + rows, :], yc,
                dimension_numbers=(((1,), (0,)), ((), ())),
                preferred_element_type=jnp.float32,
            )
            if c == ns - 1:
                for i in range(ns):
                    di = d_ref[pl.ds(i, 1)][0]
                    yi = ybf_ref[pl.ds(i * _RB, _RB), :].astype(jnp.float32)
                    zi = z2[i * _RB:(i + 1) * _RB, :]
                    ai = acc_ref[pl.ds(i * _RB, _RB), :]
                    o_ref[pl.ds(i * _RB, _RB), :] = (
                        di * (ai + zi) + di * yi + b_ref[...])
            else:
                acc_ref[0:rows, :] += z2


def kernel(x, A, W, b):
    n, din = x.shape
    dout = W.shape[0]
    ns = n // _RB
    tri_rows = _RB * ns * (ns + 1) // 2

    out = pl.pallas_call(
        _gcn_body,
        grid=(ns + 2,),
        in_specs=[
            pl.BlockSpec((_RB, n), lambda k: (jnp.clip(k - 1, 0, ns - 1), 0)),
            pl.BlockSpec((_RB, din), lambda k: (jnp.clip(k - 1, 0, ns - 1), 0)),
            pl.BlockSpec((dout, din), lambda k: (0, 0)),
            pl.BlockSpec((1, dout), lambda k: (0, 0)),
        ],
        out_specs=pl.BlockSpec((n, dout), lambda k: (0, 0)),
        out_shape=jax.ShapeDtypeStruct((n, dout), jnp.float32),
        scratch_shapes=[
            pltpu.VMEM((tri_rows, _RB), jnp.bfloat16),
            pltpu.VMEM((ns, _RB, 1), jnp.float32),
            pltpu.VMEM((n, dout), jnp.bfloat16),
            pltpu.VMEM((n, dout), jnp.float32),
        ],
    )(A, x, W, b.reshape(1, dout))
    return out
